# baseline (speedup 1.0000x reference)
import jax
import jax.numpy as jnp
from jax import lax
from jax.experimental import pallas as pl
from jax.experimental.pallas import tpu as pltpu
from jax.experimental.pallas import tpu_sc as plsc

MAXV = 100000
D = 32
HW = 256
NIDX = 3 * HW
NC, NS = 2, 16
NW = NC * NS
B = 1024
IMGS_PER_W = B // NW


def _sc_body(inputs_hbm, table_hbm, out_hbm, raw_v, idx_v, rows_v, out_v,
             gsems, osem):
    wid = lax.axis_index("s") * NC + lax.axis_index("c")
    iota = lax.iota(jnp.int32, 16)
    base = wid * IMGS_PER_W

    def stage(par, img):
        pltpu.sync_copy(inputs_hbm.at[img], raw_v.at[par])
        for c in range(3):
            off = jnp.int32(c * MAXV)

            @pl.loop(0, 16)
            def _mv(h):
                v = raw_v[par, c, h, :] + off
                idx_v[par, 2 * c + (h >> 3), pl.ds((h & 7) * 16, 16)] = v

        for j in range(6):
            pltpu.async_copy(
                table_hbm.at[idx_v.at[par, j]],
                rows_v.at[par, pl.ds(j * 128, 128)],
                gsems[par],
            )

    def wait_gathers(par):
        for j in range(6):
            pltpu.make_async_copy(
                table_hbm.at[idx_v.at[par, j]],
                rows_v.at[par, pl.ds(j * 128, 128)],
                gsems[par],
            ).wait()

    stage(0, base)

    @pl.loop(0, IMGS_PER_W // 2)
    def _pair(kk):
        for par in range(2):
            k = kk * 2 + par
            img = base + k

            @pl.when(k + 1 < IMGS_PER_W)
            def _():
                stage(1 - par, img + 1)

            wait_gathers(par)

            @pl.when(k >= 2)
            def _():
                pltpu.make_async_copy(
                    out_v.at[par], out_hbm.at[img - 2], osem
                ).wait()

            @pl.loop(0, HW)
            def _acc(p):
                s0 = pl.ds(0, 16)
                s1 = pl.ds(16, 16)
                a0 = (
                    rows_v[par, p, s0]
                    + rows_v[par, p + 256, s0]
                    + rows_v[par, p + 512, s0]
                )
                a1 = (
                    rows_v[par, p, s1]
                    + rows_v[par, p + 256, s1]
                    + rows_v[par, p + 512, s1]
                )
                h = jnp.full((16,), p >> 4, jnp.int32)
                w = jnp.full((16,), p & 15, jnp.int32)
                plsc.store_scatter(out_v.at[par], [iota, h, w], a0)
                plsc.store_scatter(out_v.at[par], [iota + 16, h, w], a1)

            pltpu.async_copy(out_v.at[par], out_hbm.at[img], osem)

    for par in range(2):
        img = base + IMGS_PER_W - 2 + par
        pltpu.make_async_copy(out_v.at[par], out_hbm.at[img], osem).wait()


@jax.jit
def _bow_embed(inputs, table):
    f = pl.kernel(
        _sc_body,
        out_type=jax.ShapeDtypeStruct((B, D, 16, 16), jnp.float32),
        mesh=plsc.VectorSubcoreMesh(core_axis_name="c", subcore_axis_name="s"),
        compiler_params=pltpu.CompilerParams(
            needs_layout_passes=False, use_tc_tiling_on_sc=False
        ),
        scratch_types=[
            pltpu.VMEM((2, 3, 16, 16), jnp.int32),
            pltpu.VMEM((2, 6, 128), jnp.int32),
            pltpu.VMEM((2, NIDX, D), jnp.float32),
            pltpu.VMEM((2, D, 16, 16), jnp.float32),
            [pltpu.SemaphoreType.DMA, pltpu.SemaphoreType.DMA],
            pltpu.SemaphoreType.DMA,
        ],
    )
    return f(inputs, table)


def kernel(inputs, table):
    return _bow_embed(inputs, table)


# speedup vs baseline: 1.1129x; 1.1129x over previous
import jax
import jax.numpy as jnp
from jax import lax
from jax.experimental import pallas as pl
from jax.experimental.pallas import tpu as pltpu
from jax.experimental.pallas import tpu_sc as plsc

MAXV = 100000
D = 32
HW = 256
NIDX = 3 * HW
NC, NS = 2, 16
NW = NC * NS
B = 1024
IMGS_PER_W = B // NW


def _sc_body(in_hbm, table_hbm, out_hbm, idx_v, rows_v, out_v, gsems, osem):
    wid = lax.axis_index("s") * NC + lax.axis_index("c")
    iota = lax.iota(jnp.int32, 16)
    base = wid * IMGS_PER_W

    def stage(par, img):
        pltpu.sync_copy(in_hbm.at[pl.ds(img * 6, 6)], idx_v.at[par])
        for r in range(6):
            off = jnp.int32((r // 2) * MAXV)

            @pl.loop(0, 8)
            def _off(i):
                sl = pl.ds(i * 16, 16)
                idx_v[par, r, sl] = idx_v[par, r, sl] + off

        for j in range(6):
            pltpu.async_copy(
                table_hbm.at[idx_v.at[par, j]],
                rows_v.at[par, pl.ds(j * 128, 128)],
                gsems[par],
            )

    def wait_gathers(par):
        for j in range(6):
            pltpu.make_async_copy(
                table_hbm.at[idx_v.at[par, j]],
                rows_v.at[par, pl.ds(j * 128, 128)],
                gsems[par],
            ).wait()

    stage(0, base)

    @pl.loop(0, IMGS_PER_W // 2)
    def _pair(kk):
        for par in range(2):
            k = kk * 2 + par
            img = base + k

            @pl.when(k + 1 < IMGS_PER_W)
            def _():
                stage(1 - par, img + 1)

            wait_gathers(par)

            @pl.when(k >= 2)
            def _():
                pltpu.make_async_copy(
                    out_v.at[par, :, pl.ds(0, 128)],
                    out_hbm.at[pl.ds((img - 2) * 64, 64)],
                    osem,
                ).wait()

            @pl.loop(0, HW)
            def _acc(p):
                s0 = pl.ds(0, 16)
                s1 = pl.ds(16, 16)
                a0 = (
                    rows_v[par, p, s0]
                    + rows_v[par, p + 256, s0]
                    + rows_v[par, p + 512, s0]
                )
                a1 = (
                    rows_v[par, p, s1]
                    + rows_v[par, p + 256, s1]
                    + rows_v[par, p + 512, s1]
                )
                row = 2 * iota + (p >> 7)
                col = jnp.full((16,), p & 127, jnp.int32)
                plsc.store_scatter(out_v.at[par], [row, col], a0)
                plsc.store_scatter(out_v.at[par], [row + 32, col], a1)

            pltpu.async_copy(
                out_v.at[par, :, pl.ds(0, 128)],
                out_hbm.at[pl.ds(img * 64, 64)],
                osem,
            )

    for par in range(2):
        img = base + IMGS_PER_W - 2 + par
        pltpu.make_async_copy(
            out_v.at[par, :, pl.ds(0, 128)],
            out_hbm.at[pl.ds(img * 64, 64)],
            osem,
        ).wait()


@jax.jit
def _bow_embed(in2d, table):
    f = pl.kernel(
        _sc_body,
        out_type=jax.ShapeDtypeStruct((B * 64, 128), jnp.float32),
        mesh=plsc.VectorSubcoreMesh(core_axis_name="c", subcore_axis_name="s"),
        compiler_params=pltpu.CompilerParams(
            needs_layout_passes=False, use_tc_tiling_on_sc=False
        ),
        scratch_types=[
            pltpu.VMEM((2, 6, 128), jnp.int32),
            pltpu.VMEM((2, NIDX, D), jnp.float32),
            pltpu.VMEM((2, 64, 129), jnp.float32),
            [pltpu.SemaphoreType.DMA, pltpu.SemaphoreType.DMA],
            pltpu.SemaphoreType.DMA,
        ],
    )
    return f(in2d, table)


def kernel(inputs, table):
    in2d = inputs.reshape(B * 6, 128)
    out = _bow_embed(in2d, table)
    return out.reshape(B, D, 16, 16)


# speedup vs baseline: 1.9485x; 1.7508x over previous
import jax
import jax.numpy as jnp
from jax import lax
from jax.experimental import pallas as pl
from jax.experimental.pallas import tpu as pltpu
from jax.experimental.pallas import tpu_sc as plsc

MAXV = 100000
D = 32
HW = 256
NIDX = 3 * HW
NC, NS = 2, 16
NW = NC * NS
B = 1024
IMGS_PER_W = B // NW


def _sc_body(in_hbm, table_hbm, out_hbm, idx_v, rows_v, out_v, gsems, osem):
    wid = lax.axis_index("s") * NC + lax.axis_index("c")
    iota = lax.iota(jnp.int32, 16)
    base = wid * IMGS_PER_W

    def stage(par, img):
        pltpu.sync_copy(in_hbm.at[pl.ds(img * 6, 6)], idx_v.at[par])
        for r in range(6):
            off = jnp.int32((r // 2) * MAXV)

            @pl.loop(0, 8)
            def _off(i):
                sl = pl.ds(i * 16, 16)
                idx_v[par, r, sl] = idx_v[par, r, sl] + off

        for j in range(6):
            pltpu.async_copy(
                table_hbm.at[idx_v.at[par, j]],
                rows_v.at[par, pl.ds(j * 128, 128)],
                gsems[par],
            )

    def wait_gathers(par):
        for j in range(6):
            pltpu.make_async_copy(
                table_hbm.at[idx_v.at[par, j]],
                rows_v.at[par, pl.ds(j * 128, 128)],
                gsems[par],
            ).wait()

    stage(0, base)

    @pl.loop(0, IMGS_PER_W // 2)
    def _pair(kk):
        for par in range(2):
            k = kk * 2 + par
            img = base + k

            @pl.when(k + 1 < IMGS_PER_W)
            def _():
                stage(1 - par, img + 1)

            wait_gathers(par)

            @pl.when(k >= 2)
            def _():
                pltpu.make_async_copy(
                    out_v.at[par, :, pl.ds(0, 128)],
                    out_hbm.at[img - 2],
                    osem,
                ).wait()

            @pl.loop(0, HW)
            def _acc(p):
                s0 = pl.ds(0, 16)
                s1 = pl.ds(16, 16)
                a0 = (
                    rows_v[par, p, s0]
                    + rows_v[par, p + 256, s0]
                    + rows_v[par, p + 512, s0]
                )
                a1 = (
                    rows_v[par, p, s1]
                    + rows_v[par, p + 256, s1]
                    + rows_v[par, p + 512, s1]
                )
                row = 2 * iota + (p >> 7)
                col = jnp.full((16,), p & 127, jnp.int32)
                plsc.store_scatter(out_v.at[par], [row, col], a0)
                plsc.store_scatter(out_v.at[par], [row + 32, col], a1)

            pltpu.async_copy(
                out_v.at[par, :, pl.ds(0, 128)],
                out_hbm.at[img],
                osem,
            )

    for par in range(2):
        img = base + IMGS_PER_W - 2 + par
        pltpu.make_async_copy(
            out_v.at[par, :, pl.ds(0, 128)],
            out_hbm.at[img],
            osem,
        ).wait()


@jax.jit
def _bow_embed(in2d, table):
    f = pl.kernel(
        _sc_body,
        out_type=jax.ShapeDtypeStruct((B, 64, 128), jnp.float32),
        mesh=plsc.VectorSubcoreMesh(core_axis_name="c", subcore_axis_name="s"),
        compiler_params=pltpu.CompilerParams(
            needs_layout_passes=False, use_tc_tiling_on_sc=False
        ),
        scratch_types=[
            pltpu.VMEM((2, 6, 128), jnp.int32),
            pltpu.VMEM((2, NIDX, D), jnp.float32),
            pltpu.VMEM((2, 64, 129), jnp.float32),
            [pltpu.SemaphoreType.DMA, pltpu.SemaphoreType.DMA],
            pltpu.SemaphoreType.DMA,
        ],
    )
    return f(in2d, table)


def kernel(inputs, table):
    in2d = inputs.reshape(B * 6, 128)
    out = _bow_embed(in2d, table)
    return out.reshape(B, D, 16, 16)
